# baseline (speedup 1.0000x reference)
import functools
import math

import jax
import jax.numpy as jnp
from jax import lax
from jax.experimental import pallas as pl
from jax.experimental.pallas import tpu as pltpu
from jax.experimental.pallas import tpu_sc as plsc

D_MODEL = 64
SCALE = math.sqrt(D_MODEL)

NC = 2
NS = 16
NW = NC * NS
L = 16

CB = 128


def _make_emb_kernel(B, n_chunks):
    b_per_w = n_chunks * CB
    mesh = plsc.VectorSubcoreMesh(core_axis_name="c", subcore_axis_name="s")

    @functools.partial(
        pl.kernel,
        mesh=mesh,
        compiler_params=pltpu.CompilerParams(use_tc_tiling_on_sc=False),
        out_type=jax.ShapeDtypeStruct((B, D_MODEL), jnp.float32),
        scratch_types=[
            pltpu.VMEM((n_chunks, CB), jnp.int32),
            pltpu.VMEM((CB, D_MODEL), jnp.float32),
            pltpu.SemaphoreType.DMA,
        ],
    )
    def emb(x_hbm, table_hbm, out_hbm, idx_v, rows_v, gsem):
        wid = lax.axis_index("s") * NC + lax.axis_index("c")
        base = wid * b_per_w
        pltpu.sync_copy(x_hbm.at[wid], idx_v)

        def chunk_body(g, carry):
            pltpu.async_copy(table_hbm.at[idx_v.at[g]], rows_v, gsem).wait()

            def scale_row(i, c2):
                for c in range(D_MODEL // L):
                    sl = pl.ds(c * L, L)
                    rows_v[i, sl] = rows_v[i, sl] * SCALE
                return c2

            lax.fori_loop(0, CB, scale_row, 0)

            pltpu.sync_copy(rows_v, out_hbm.at[pl.ds(base + g * CB, CB)])
            return carry

        lax.fori_loop(0, n_chunks, chunk_body, 0)

    return emb


def kernel(x, table):
    B0, B1 = x.shape
    B = B0 * B1
    assert B % (NW * CB) == 0
    n_chunks = B // (NW * CB)
    xw = x.reshape(NW, n_chunks, CB).astype(jnp.int32)
    emb = _make_emb_kernel(B, n_chunks)
    out = emb(xw, table)
    return out.reshape(B0, B1, D_MODEL)


# speedup vs baseline: 1.1050x; 1.1050x over previous
import functools
import math

import jax
import jax.numpy as jnp
from jax import lax
from jax.experimental import pallas as pl
from jax.experimental.pallas import tpu as pltpu
from jax.experimental.pallas import tpu_sc as plsc

D_MODEL = 64
SCALE = math.sqrt(D_MODEL)

NC = 2
NS = 16
NW = NC * NS
L = 16

CB = 128
NBUF = 4
LEAD = 2


def _make_emb_kernel(B, n_chunks):
    b_per_w = n_chunks * CB
    mesh = plsc.VectorSubcoreMesh(core_axis_name="c", subcore_axis_name="s")

    @functools.partial(
        pl.kernel,
        mesh=mesh,
        compiler_params=pltpu.CompilerParams(use_tc_tiling_on_sc=False),
        out_type=jax.ShapeDtypeStruct((B, D_MODEL), jnp.float32),
        scratch_types=[
            pltpu.VMEM((n_chunks, CB), jnp.int32),
            pltpu.VMEM((NBUF, CB, D_MODEL), jnp.float32),
        ]
        + [pltpu.SemaphoreType.DMA] * (2 * NBUF),
    )
    def emb(x_hbm, table_hbm, out_hbm, idx_v, rows_v, *sems):
        gsems = sems[:NBUF]
        ssems = sems[NBUF:]
        wid = lax.axis_index("s") * NC + lax.axis_index("c")
        base = wid * b_per_w
        pltpu.sync_copy(x_hbm.at[wid], idx_v)

        def start_gather(g, slot):
            pltpu.make_async_copy(
                table_hbm.at[idx_v.at[g]], rows_v.at[slot], gsems[slot]
            ).start()

        def wait_gather(g, slot):
            pltpu.make_async_copy(
                table_hbm.at[idx_v.at[g]], rows_v.at[slot], gsems[slot]
            ).wait()

        def start_scatter(g, slot):
            pltpu.make_async_copy(
                rows_v.at[slot], out_hbm.at[pl.ds(base + g * CB, CB)], ssems[slot]
            ).start()

        def wait_scatter(slot):
            pltpu.make_async_copy(
                rows_v.at[slot], out_hbm.at[pl.ds(base, CB)], ssems[slot]
            ).wait()

        def scale(slot):
            r = rows_v.at[slot]

            @plsc.parallel_loop(0, CB, 4, unroll=2)
            def _(i):
                for rr in range(4):
                    for c in range(D_MODEL // L):
                        sl = pl.ds(c * L, L)
                        r[i + rr, sl] = r[i + rr, sl] * SCALE

        def process(g, slot):
            wait_gather(g, slot)
            scale(slot)
            start_scatter(g, slot)

        start_gather(0, 0)
        start_gather(1, 1)
        process(0, 0)
        start_gather(2, 2)
        process(1, 1)
        start_gather(3, 3)

        def outer(go, carry):
            g0 = 2 + go * NBUF
            for bi in range(NBUF):
                g = g0 + bi
                slot = (2 + bi) % NBUF
                nslot = (slot + LEAD) % NBUF
                process(g, slot)
                wait_scatter(nslot)
                start_gather(g + LEAD, nslot)
            return carry

        lax.fori_loop(0, (n_chunks - NBUF) // NBUF, outer, 0)

        process(n_chunks - 2, (n_chunks - 2) % NBUF)
        process(n_chunks - 1, (n_chunks - 1) % NBUF)
        for s in range(NBUF):
            wait_scatter(s)

    return emb


def kernel(x, table):
    B0, B1 = x.shape
    B = B0 * B1
    assert B % (NW * CB) == 0
    n_chunks = B // (NW * CB)
    assert n_chunks % NBUF == 0 and n_chunks >= 2 * NBUF
    xw = x.reshape(NW, n_chunks, CB).astype(jnp.int32)
    emb = _make_emb_kernel(B, n_chunks)
    out = emb(xw, table)
    return out.reshape(B0, B1, D_MODEL)
